# Initial kernel scaffold; baseline (speedup 1.0000x reference)
import jax
import jax.numpy as jnp
from jax.experimental import pallas as pl


def kernel(item_seq, user_ids, items_to_predict, U, E, Q, Qb, W_fi, b_fi, W_fu, b_fu, gate_item, gate_user):
    raise NotImplementedError("write your pallas kernel here")


# speedup vs baseline: 4.7523x; 4.7523x over previous
import functools

import jax
import jax.numpy as jnp
from jax import lax
from jax.experimental import pallas as pl
from jax.experimental.pallas import tpu as pltpu
from jax.experimental.pallas import tpu_sc as plsc

DIMS = 64
GW = 128


def _sc_gather_body(E_hbm, Q_hbm, U_hbm, seq_hbm, pred_hbm, uid_hbm,
                    ie_hbm, w2_hbm, ue_hbm):
    n = seq_hbm.shape[1]

    def body(seq_v, pred_v, ie_v, w2_v):
        pltpu.sync_copy(E_hbm.at[seq_v.at[0]], ie_v)
        pltpu.sync_copy(Q_hbm.at[pred_v.at[0]], w2_v)

    pltpu.emit_pipeline(
        body,
        grid=(n // GW,),
        in_specs=[pl.BlockSpec((1, GW), lambda i: (0, i)),
                  pl.BlockSpec((1, GW), lambda i: (0, i))],
        out_specs=[pl.BlockSpec((GW, DIMS), lambda i: (i, 0)),
                   pl.BlockSpec((GW, DIMS), lambda i: (i, 0))],
        core_axis_name=("c", "s"),
        dimension_semantics=(pltpu.PARALLEL,),
    )(seq_hbm, pred_hbm, ie_hbm, w2_hbm)

    nb = uid_hbm.shape[1]

    def ubody(uid_v, ue_v):
        pltpu.sync_copy(U_hbm.at[uid_v.at[0]], ue_v)

    pltpu.emit_pipeline(
        ubody,
        grid=(nb // GW,),
        in_specs=[pl.BlockSpec((1, GW), lambda i: (0, i))],
        out_specs=[pl.BlockSpec((GW, DIMS), lambda i: (i, 0))],
        core_axis_name=("c", "s"),
        dimension_semantics=(pltpu.PARALLEL,),
    )(uid_hbm, ue_hbm)


def _sc_gather(E, Q, U, seq_t, pred, uid):
    n = seq_t.shape[1]
    nb = uid.shape[1]
    mesh = plsc.VectorSubcoreMesh(core_axis_name="c", subcore_axis_name="s")
    k = pl.kernel(
        _sc_gather_body,
        out_type=[
            jax.ShapeDtypeStruct((n, DIMS), jnp.float32),
            jax.ShapeDtypeStruct((n, DIMS), jnp.float32),
            jax.ShapeDtypeStruct((nb, DIMS), jnp.float32),
        ],
        mesh=mesh,
        compiler_params=pltpu.CompilerParams(use_tc_tiling_on_sc=False),
    )
    return k(E, Q, U, seq_t, pred, uid)


def _sc_qb_body(n_workers, Qb_hbm, pred_hbm, b2_hbm, qb_v, idx_v, out_v):
    wid = lax.axis_index("s") * 2 + lax.axis_index("c")
    per = pred_hbm.shape[1] // n_workers
    base = wid * per
    pltpu.sync_copy(Qb_hbm, qb_v)
    pltpu.sync_copy(pred_hbm.at[0, pl.ds(base, per)], idx_v)

    @pl.loop(0, per, step=16)
    def _(j):
        out_v[pl.ds(j, 16)] = plsc.load_gather(qb_v, [idx_v[pl.ds(j, 16)]])

    pltpu.sync_copy(out_v, b2_hbm.at[pl.ds(base, per)])


def _sc_qb_gather(Qb_flat, pred):
    n = pred.shape[1]
    nv = Qb_flat.shape[0]
    mesh = plsc.VectorSubcoreMesh(core_axis_name="c", subcore_axis_name="s")
    nw = 32
    k = pl.kernel(
        functools.partial(_sc_qb_body, nw),
        out_type=jax.ShapeDtypeStruct((n,), jnp.float32),
        mesh=mesh,
        scratch_types=[
            pltpu.VMEM((nv,), jnp.float32),
            pltpu.VMEM((n // nw,), jnp.int32),
            pltpu.VMEM((n // nw,), jnp.float32),
        ],
        compiler_params=pltpu.CompilerParams(use_tc_tiling_on_sc=False,
                                             needs_layout_passes=False),
    )
    return k(Qb_flat, pred)


def _tc_body(L, ie_ref, ue_ref, w2_ref, b2_ref, wfiT_ref, wfuT_ref, bias_ref,
             giT_ref, guT_ref, out_ref, ulin_s, accu_s, acci_s, accs_s):
    l = pl.program_id(1)
    ie = ie_ref[0]
    ue = ue_ref[...]

    @pl.when(l == 0)
    def _():
        ulin_s[...] = (jnp.dot(ue, wfuT_ref[...],
                               preferred_element_type=jnp.float32)
                       + bias_ref[...])

    gin = jnp.dot(ie, wfiT_ref[...], preferred_element_type=jnp.float32)
    gate = jax.nn.sigmoid(gin + ulin_s[...])
    gated = ie * gate
    s1 = jnp.sum(gated * giT_ref[...], axis=1, keepdims=True)
    s2 = jnp.sum(ue * guT_ref[0], axis=1, keepdims=True)
    score = jax.nn.sigmoid(s1 + s2)
    u_c = gated * score

    @pl.when(l == 0)
    def _():
        accu_s[...] = u_c
        acci_s[...] = ie
        accs_s[...] = score

    @pl.when(l > 0)
    def _():
        accu_s[...] += u_c
        acci_s[...] += ie
        accs_s[...] += score

    @pl.when(l == L - 1)
    def _():
        v = ue + accu_s[...] / accs_s[...] + acci_s[...]
        res = b2_ref[...] + jnp.sum(w2_ref[...] * v[:, None, :], axis=2)
        out_ref[...] = res


def _tc_compute(ie_t, ue, w2, b2, wfiT, wfuT, bias, giT, guT3, Bb=512):
    Lk, B, D = ie_t.shape
    T = w2.shape[1]
    grid = (B // Bb, Lk)
    return pl.pallas_call(
        functools.partial(_tc_body, Lk),
        grid=grid,
        in_specs=[
            pl.BlockSpec((1, Bb, D), lambda i, l: (l, i, 0)),
            pl.BlockSpec((Bb, D), lambda i, l: (i, 0)),
            pl.BlockSpec((Bb, T, D), lambda i, l: (i, 0, 0)),
            pl.BlockSpec((Bb, T), lambda i, l: (i, 0)),
            pl.BlockSpec((D, D), lambda i, l: (0, 0)),
            pl.BlockSpec((D, D), lambda i, l: (0, 0)),
            pl.BlockSpec((1, D), lambda i, l: (0, 0)),
            pl.BlockSpec((1, D), lambda i, l: (0, 0)),
            pl.BlockSpec((1, 1, D), lambda i, l: (l, 0, 0)),
        ],
        out_specs=pl.BlockSpec((Bb, T), lambda i, l: (i, 0)),
        out_shape=jax.ShapeDtypeStruct((B, T), jnp.float32),
        scratch_shapes=[
            pltpu.VMEM((Bb, D), jnp.float32),
            pltpu.VMEM((Bb, D), jnp.float32),
            pltpu.VMEM((Bb, D), jnp.float32),
            pltpu.VMEM((Bb, 1), jnp.float32),
        ],
    )(ie_t, ue, w2, b2, wfiT, wfuT, bias, giT, guT3)


def kernel(item_seq, user_ids, items_to_predict, U, E, Q, Qb, W_fi, b_fi,
           W_fu, b_fu, gate_item, gate_user):
    B, L = item_seq.shape
    T = items_to_predict.shape[1]
    seq_t = jnp.transpose(item_seq).astype(jnp.int32).reshape(1, L * B)
    pred = items_to_predict.astype(jnp.int32).reshape(1, B * T)
    uid = user_ids.astype(jnp.int32).reshape(1, B)

    ie_flat, w2_flat, ue = _sc_gather(E, Q, U, seq_t, pred, uid)
    b2_flat = _sc_qb_gather(Qb.reshape(-1), pred)

    ie_t = ie_flat.reshape(L, B, DIMS)
    w2 = w2_flat.reshape(B, T, DIMS)
    b2 = b2_flat.reshape(B, T)
    bias = (b_fi + b_fu).reshape(1, DIMS)
    giT = gate_item.T
    guT3 = gate_user.T.reshape(L, 1, DIMS)
    return _tc_compute(ie_t, ue, w2, b2, jnp.transpose(W_fi),
                       jnp.transpose(W_fu), bias, giT, guT3)
